# speedup vs baseline: 3.3085x; 3.3085x over previous
import jax
import jax.numpy as jnp
from jax.experimental import pallas as pl
from jax.experimental.pallas import tpu as pltpu
from jax.experimental.pallas import tpu_sc as plsc

VOCAB = 1000000
HIDDEN = 128
SEQ = 50
EPS = 1e-12

GATHER_WINDOW = 128


def _sc_gather(word_emb, ids_flat, n_rows):
    vector_mesh = plsc.VectorSubcoreMesh(
        core_axis_name="core", subcore_axis_name="subcore"
    )

    @pl.kernel(
        out_type=jax.ShapeDtypeStruct((n_rows, HIDDEN), word_emb.dtype),
        mesh=vector_mesh,
    )
    def gather_kernel(x_hbm, i_hbm, o_hbm):
        def body(i_vmem, o_vmem):
            pltpu.sync_copy(x_hbm.at[i_vmem.at[0]], o_vmem)

        pltpu.emit_pipeline(
            body,
            grid=(n_rows // GATHER_WINDOW,),
            in_specs=[
                pl.BlockSpec((1, GATHER_WINDOW), index_map=lambda i: (0, i))
            ],
            out_specs=[
                pl.BlockSpec((GATHER_WINDOW, HIDDEN), index_map=lambda i: (i, 0))
            ],
            core_axis_name=("core", "subcore"),
            dimension_semantics=(pltpu.PARALLEL,),
        )(i_hbm, o_hbm)

    return gather_kernel(word_emb, ids_flat)


def _tc_add_ln_kernel(x_ref, pos_ref, type_ref, gamma_ref, beta_ref, o_ref):
    x = x_ref[...]
    comb = pos_ref[...] + type_ref[0:1, :]
    e = x + comb[None, :, :]
    mean = jnp.mean(e, axis=-1, keepdims=True)
    cent = e - mean
    var = jnp.mean(cent * cent, axis=-1, keepdims=True)
    inv = jax.lax.rsqrt(var + EPS)
    o_ref[...] = cent * inv * gamma_ref[...][None, :, :] + beta_ref[...][None, :, :]


def _tc_add_ln(gathered, pos50, type_emb, gamma, beta, batch):
    b_blk = 128
    grid = (batch // b_blk,)
    return pl.pallas_call(
        _tc_add_ln_kernel,
        grid=grid,
        in_specs=[
            pl.BlockSpec((b_blk, SEQ, HIDDEN), lambda i: (i, 0, 0)),
            pl.BlockSpec((SEQ, HIDDEN), lambda i: (0, 0)),
            pl.BlockSpec((2, HIDDEN), lambda i: (0, 0)),
            pl.BlockSpec((1, HIDDEN), lambda i: (0, 0)),
            pl.BlockSpec((1, HIDDEN), lambda i: (0, 0)),
        ],
        out_specs=pl.BlockSpec((b_blk, SEQ, HIDDEN), lambda i: (i, 0, 0)),
        out_shape=jax.ShapeDtypeStruct((batch, SEQ, HIDDEN), jnp.float32),
    )(gathered, pos50, type_emb, gamma, beta)


def kernel(input_ids, word_emb, pos_emb, type_emb, gamma, beta):
    batch, seq = input_ids.shape
    n_rows = batch * seq
    ids_flat = input_ids.astype(jnp.int32).reshape(1, n_rows)
    gathered = _sc_gather(word_emb, ids_flat, n_rows)
    gathered = gathered.reshape(batch, seq, HIDDEN)
    pos50 = pos_emb[:seq]
    gamma2 = gamma.reshape(1, HIDDEN)
    beta2 = beta.reshape(1, HIDDEN)
    return _tc_add_ln(gathered, pos50, type_emb, gamma2, beta2, batch)


# speedup vs baseline: 4.1550x; 1.2558x over previous
import jax
import jax.numpy as jnp
from jax.experimental import pallas as pl
from jax.experimental.pallas import tpu as pltpu
from jax.experimental.pallas import tpu_sc as plsc

VOCAB = 1000000
HIDDEN = 128
SEQ = 50
EPS = 1e-12

GATHER_WINDOW = 128


def _sc_gather(word_emb, ids_flat, n_rows):
    vector_mesh = plsc.VectorSubcoreMesh(
        core_axis_name="core", subcore_axis_name="subcore"
    )

    @pl.kernel(
        out_type=jax.ShapeDtypeStruct((n_rows, HIDDEN), word_emb.dtype),
        mesh=vector_mesh,
    )
    def gather_kernel(x_hbm, i_hbm, o_hbm):
        def body(i_vmem, o_vmem):
            pltpu.sync_copy(x_hbm.at[i_vmem.at[0]], o_vmem)

        pltpu.emit_pipeline(
            body,
            grid=(n_rows // GATHER_WINDOW,),
            in_specs=[
                pl.BlockSpec((1, GATHER_WINDOW), index_map=lambda i: (0, i))
            ],
            out_specs=[
                pl.BlockSpec((GATHER_WINDOW, HIDDEN), index_map=lambda i: (i, 0))
            ],
            core_axis_name=("core", "subcore"),
            dimension_semantics=(pltpu.PARALLEL,),
        )(i_hbm, o_hbm)

    return gather_kernel(word_emb, ids_flat)


B_BLK = 64


def _tc_add_ln_kernel(x_ref, comb_ref, gamma_ref, beta_ref, o_ref):
    x = x_ref[...]
    e = x + comb_ref[...]
    mean = jnp.mean(e, axis=-1, keepdims=True)
    cent = e - mean
    var = jnp.mean(cent * cent, axis=-1, keepdims=True)
    inv = jax.lax.rsqrt(var + EPS)
    y = cent * inv * gamma_ref[...] + beta_ref[...]
    o_ref[...] = y.reshape(B_BLK, SEQ, HIDDEN)


def _tc_add_ln(gathered, comb_tiled, gamma, beta, batch):
    rows_blk = B_BLK * SEQ
    grid = (batch // B_BLK,)
    return pl.pallas_call(
        _tc_add_ln_kernel,
        grid=grid,
        in_specs=[
            pl.BlockSpec((rows_blk, HIDDEN), lambda i: (i, 0)),
            pl.BlockSpec((rows_blk, HIDDEN), lambda i: (0, 0)),
            pl.BlockSpec((1, HIDDEN), lambda i: (0, 0)),
            pl.BlockSpec((1, HIDDEN), lambda i: (0, 0)),
        ],
        out_specs=pl.BlockSpec((B_BLK, SEQ, HIDDEN), lambda i: (i, 0, 0)),
        out_shape=jax.ShapeDtypeStruct((batch, SEQ, HIDDEN), jnp.float32),
    )(gathered, comb_tiled, gamma, beta)


def kernel(input_ids, word_emb, pos_emb, type_emb, gamma, beta):
    batch, seq = input_ids.shape
    n_rows = batch * seq
    ids_flat = input_ids.astype(jnp.int32).reshape(1, n_rows)
    gathered = _sc_gather(word_emb, ids_flat, n_rows)
    comb_tiled = jnp.tile(pos_emb[:seq] + type_emb[0:1, :], (B_BLK, 1))
    gamma2 = gamma.reshape(1, HIDDEN)
    beta2 = beta.reshape(1, HIDDEN)
    return _tc_add_ln(gathered, comb_tiled, gamma2, beta2, batch)
